# baseline (speedup 1.0000x reference)
import math

import jax
import jax.numpy as jnp
from jax.experimental import pallas as pl
from jax.experimental.pallas import tpu as pltpu

_D = 4096
_B = 8
_HID = 64
_TDIM = 128
_BN = 256
_NB = _D // _BN
_NSUB = 4
_SUB = _BN // _NSUB
_F32 = jnp.float32
_BF16 = jnp.bfloat16
_HI = jax.lax.Precision.HIGHEST


def _coef(w0, w1, i, j):
    return jnp.sum(w0[:, i:i + 1] * w1[:, j:j + 1], axis=0, keepdims=True)


def _body(ld_hbm, lu_hbm, xt_ref, t_ref, freqs_ref, tw1_ref, tb1_ref,
          tw2_ref, tb2_ref, w0_ref, w1_ref, mapwt_ref, mapb_ref,
          ow1_ref, ob1_ref, ow2_ref, ob2_ref,
          out_ref,
          stash_ref, bufa_ref, bufb_ref, u_ref, r_ref, p_ref, q_ref,
          xo_ref, sema, semb):

    def start(src, chunk, buf, sem):
        for s in range(_NSUB):
            pltpu.make_async_copy(
                src.at[pl.ds(chunk * _BN + s * _SUB, _SUB), :],
                buf.at[pl.ds(s * _SUB, _SUB), :], sem.at[s]).start()

    def wait(src, chunk, buf, sem):
        for s in range(_NSUB):
            pltpu.make_async_copy(
                src.at[pl.ds(chunk * _BN + s * _SUB, _SUB), :],
                buf.at[pl.ds(s * _SUB, _SUB), :], sem.at[s]).wait()

    start(ld_hbm, 0, bufa_ref, sema)
    start(ld_hbm, 1, bufb_ref, semb)

    args = t_ref[...] * freqs_ref[...]
    cosr = jnp.cos(args)
    sinr = jnp.sin(args)
    h = jnp.dot(cosr, tw1_ref[0:_TDIM // 2, :], preferred_element_type=_F32)
    h = h + jnp.dot(sinr, tw1_ref[_TDIM // 2:_TDIM, :], preferred_element_type=_F32)
    h = h + tb1_ref[...]
    h = h * jax.lax.logistic(h)
    tout = jnp.dot(h, tw2_ref[...], preferred_element_type=_F32) + tb2_ref[...]
    w0 = w0_ref[...]
    w1 = w1_ref[...]
    c1 = _coef(w0, w1, 0, 0)
    c2 = _coef(w0, w1, 1, 0) + _coef(w0, w1, 0, 1)
    c3 = _coef(w0, w1, 2, 0) + _coef(w0, w1, 0, 2)
    c4 = _coef(w0, w1, 1, 1)
    c5 = _coef(w0, w1, 2, 1)
    c6 = _coef(w0, w1, 1, 2)
    c7 = _coef(w0, w1, 2, 2)

    xt = xt_ref[...]
    half_nb = _NB // 2

    def phase(src, nxt_src, consume, last):
        def one(chunk, buf, sem):
            wait(src, chunk, buf, sem)
            consume(chunk, buf)
            nxt = chunk + 2

            @pl.when(nxt < _NB)
            def _pref():
                start(src, nxt, buf, sem)

            if not last:
                @pl.when(nxt >= _NB)
                def _pref2():
                    start(nxt_src, nxt - _NB, buf, sem)

        def body(k, carry):
            c0 = pl.multiple_of(2 * k, 2)
            one(c0, bufa_ref, sema)
            one(c0 + 1, bufb_ref, semb)
            return carry

        jax.lax.fori_loop(0, half_nb, body, 0, unroll=False)

    def consume0(c, buf):
        base = pl.multiple_of(c * _BN, _BN)
        rows = pl.ds(base, _BN)
        blk = buf[...]
        u_ref[rows, :] = jax.lax.dot(blk, xt, precision=_HI,
                                     preferred_element_type=_F32)
        stash_ref[rows, :] = blk.astype(_BF16)

    def consume1(c, buf):
        base = pl.multiple_of(c * _BN, _BN)
        rows = pl.ds(base, _BN)
        v = jax.lax.dot(buf[...], xt, precision=_HI, preferred_element_type=_F32)
        u = u_ref[rows, :]
        r_ref[rows, :] = (c2 * u + c3 * v).astype(_BF16)
        p_ref[rows, :] = (c4 * u + c5 * v).astype(_BF16)
        q_ref[rows, :] = c6 * u + c7 * v

    def consume2(c, buf):
        base = pl.multiple_of(c * _BN, _BN)
        rows = pl.ds(base, _BN)
        sd = jnp.dot(stash_ref[rows, :], p_ref[...],
                     preferred_element_type=_F32)
        su = jax.lax.dot(buf[...], q_ref[...], precision=_HI,
                         preferred_element_type=_F32)
        xo_ref[rows, :] = (c1 * xt_ref[rows, :]
                           + r_ref[rows, :].astype(_F32) + sd + su)

    phase(ld_hbm, lu_hbm, consume0, last=False)
    phase(lu_hbm, lu_hbm, consume1, last=False)
    phase(lu_hbm, lu_hbm, consume2, last=True)

    x2t = jax.lax.dot(mapwt_ref[...], xo_ref[...], precision=_HI,
                      preferred_element_type=_F32)
    h = jnp.transpose(x2t) + mapb_ref[...] + tout
    h = jnp.dot(h, ow1_ref[...], preferred_element_type=_F32) + ob1_ref[...]
    h = h * jax.lax.logistic(h)
    out_ref[...] = jnp.dot(h, ow2_ref[...], preferred_element_type=_F32) + ob2_ref[...]


def kernel(x, t, lap_down, lap_up, tW1, tb1, tW2, tb2, snn_w0, snn_w1,
           mapW, mapb, outW1, outb1, outW2, outb2):
    xt = x.T
    t2 = t.reshape(_B, 1)
    half = _TDIM // 2
    freqs = jnp.exp(
        -math.log(10000.0) * jnp.arange(0, half, dtype=_F32) / half
    ).reshape(1, half)
    w0r = snn_w0[0]
    w1r = snn_w1[:, 0, :]
    tb1r = tb1.reshape(1, _HID)
    tb2r = tb2.reshape(1, _HID)
    mapbr = mapb.reshape(1, _HID)
    ob1r = outb1.reshape(1, _HID)
    ob2r = outb2.reshape(1, _D)
    mapwt = mapW.T

    any_spec = pl.BlockSpec(memory_space=pl.ANY)
    vmem = pl.BlockSpec(memory_space=pltpu.VMEM)

    out = pl.pallas_call(
        _body,
        compiler_params=pltpu.CompilerParams(vmem_limit_bytes=63 << 20),
        in_specs=[any_spec, any_spec] + [vmem] * 15,
        out_specs=vmem,
        out_shape=jax.ShapeDtypeStruct((_B, _D), _F32),
        scratch_shapes=[
            pltpu.VMEM((_D, _D), _BF16),
            pltpu.VMEM((_BN, _D), _F32),
            pltpu.VMEM((_BN, _D), _F32),
            pltpu.VMEM((_D, _B), _F32),
            pltpu.VMEM((_D, _B), _BF16),
            pltpu.VMEM((_D, _B), _BF16),
            pltpu.VMEM((_D, _B), _F32),
            pltpu.VMEM((_D, _B), _F32),
            pltpu.SemaphoreType.DMA((_NSUB,)),
            pltpu.SemaphoreType.DMA((_NSUB,)),
        ],
    )(lap_down, lap_up, xt, t2, freqs, tW1, tb1r, tW2, tb2r, w0r, w1r,
      mapwt, mapbr, outW1, ob1r, outW2, ob2r)
    return out


# speedup vs baseline: 3.0606x; 3.0606x over previous
import math

import jax
import jax.numpy as jnp
from jax.experimental import pallas as pl
from jax.experimental.pallas import tpu as pltpu

_D = 4096
_B = 8
_HID = 64
_TDIM = 128
_BN = 256
_NB = _D // _BN
_NSUB = 4
_SUB = _BN // _NSUB
_F32 = jnp.float32
_BF16 = jnp.bfloat16


def _coef(w0, w1, i, j):
    return jnp.sum(w0[:, i:i + 1] * w1[:, j:j + 1], axis=0, keepdims=True)


def _body(ld_hbm, lu_hbm, xt_ref, t_ref, freqs_ref, tw1_ref, tb1_ref,
          tw2_ref, tb2_ref, w0_ref, w1_ref, mapwt_ref, mapb_ref,
          ow1_ref, ob1_ref, ow2_ref, ob2_ref,
          out_ref,
          stash_ref, bufa_ref, bufb_ref, u_ref, r_ref, p_ref, q_ref,
          xo_ref, sema, semb):

    def start(src, chunk, buf, sem):
        for s in range(_NSUB):
            pltpu.make_async_copy(
                src.at[pl.ds(chunk * _BN + s * _SUB, _SUB), :],
                buf.at[pl.ds(s * _SUB, _SUB), :], sem.at[s]).start()

    def wait(src, chunk, buf, sem):
        for s in range(_NSUB):
            pltpu.make_async_copy(
                src.at[pl.ds(chunk * _BN + s * _SUB, _SUB), :],
                buf.at[pl.ds(s * _SUB, _SUB), :], sem.at[s]).wait()

    start(ld_hbm, 0, bufa_ref, sema)
    start(ld_hbm, 1, bufb_ref, semb)

    args = t_ref[...] * freqs_ref[...]
    cosr = jnp.cos(args)
    sinr = jnp.sin(args)
    h = jnp.dot(cosr, tw1_ref[0:_TDIM // 2, :], preferred_element_type=_F32)
    h = h + jnp.dot(sinr, tw1_ref[_TDIM // 2:_TDIM, :], preferred_element_type=_F32)
    h = h + tb1_ref[...]
    h = h * jax.lax.logistic(h)
    tout = jnp.dot(h, tw2_ref[...], preferred_element_type=_F32) + tb2_ref[...]
    w0 = w0_ref[...]
    w1 = w1_ref[...]
    c1 = _coef(w0, w1, 0, 0)
    c2 = _coef(w0, w1, 1, 0) + _coef(w0, w1, 0, 1)
    c3 = _coef(w0, w1, 2, 0) + _coef(w0, w1, 0, 2)
    c4 = _coef(w0, w1, 1, 1)
    c5 = _coef(w0, w1, 2, 1)
    c6 = _coef(w0, w1, 1, 2)
    c7 = _coef(w0, w1, 2, 2)

    xt = xt_ref[...]
    half_nb = _NB // 2

    def phase(src, nxt_src, consume, last):
        def one(chunk, buf, sem):
            wait(src, chunk, buf, sem)
            consume(chunk, buf)
            nxt = chunk + 2

            @pl.when(nxt < _NB)
            def _pref():
                start(src, nxt, buf, sem)

            if not last:
                @pl.when(nxt >= _NB)
                def _pref2():
                    start(nxt_src, nxt - _NB, buf, sem)

        def body(k, carry):
            c0 = pl.multiple_of(2 * k, 2)
            one(c0, bufa_ref, sema)
            one(c0 + 1, bufb_ref, semb)
            return carry

        jax.lax.fori_loop(0, half_nb, body, 0, unroll=False)

    def consume0(c, buf):
        base = pl.multiple_of(c * _BN, _BN)
        rows = pl.ds(base, _BN)
        blk = buf[...]
        u_ref[rows, :] = jnp.dot(blk, xt, preferred_element_type=_F32)
        stash_ref[rows, :] = blk.astype(_BF16)

    def consume1(c, buf):
        base = pl.multiple_of(c * _BN, _BN)
        rows = pl.ds(base, _BN)
        v = jnp.dot(buf[...], xt, preferred_element_type=_F32)
        u = u_ref[rows, :]
        r_ref[rows, :] = (c2 * u + c3 * v).astype(_BF16)
        p_ref[rows, :] = (c4 * u + c5 * v).astype(_BF16)
        q_ref[rows, :] = c6 * u + c7 * v

    def consume2(c, buf):
        base = pl.multiple_of(c * _BN, _BN)
        rows = pl.ds(base, _BN)
        sd = jnp.dot(stash_ref[rows, :], p_ref[...],
                     preferred_element_type=_F32)
        su = jnp.dot(buf[...], q_ref[...], preferred_element_type=_F32)
        xo_ref[rows, :] = (c1 * xt_ref[rows, :]
                           + r_ref[rows, :].astype(_F32) + sd + su)

    phase(ld_hbm, lu_hbm, consume0, last=False)
    phase(lu_hbm, lu_hbm, consume1, last=False)
    phase(lu_hbm, lu_hbm, consume2, last=True)

    x2t = jnp.dot(mapwt_ref[...], xo_ref[...],
                  preferred_element_type=_F32)
    h = jnp.transpose(x2t) + mapb_ref[...] + tout
    h = jnp.dot(h, ow1_ref[...], preferred_element_type=_F32) + ob1_ref[...]
    h = h * jax.lax.logistic(h)
    out_ref[...] = jnp.dot(h, ow2_ref[...], preferred_element_type=_F32) + ob2_ref[...]


def kernel(x, t, lap_down, lap_up, tW1, tb1, tW2, tb2, snn_w0, snn_w1,
           mapW, mapb, outW1, outb1, outW2, outb2):
    xt = x.T
    t2 = t.reshape(_B, 1)
    half = _TDIM // 2
    freqs = jnp.exp(
        -math.log(10000.0) * jnp.arange(0, half, dtype=_F32) / half
    ).reshape(1, half)
    w0r = snn_w0[0]
    w1r = snn_w1[:, 0, :]
    tb1r = tb1.reshape(1, _HID)
    tb2r = tb2.reshape(1, _HID)
    mapbr = mapb.reshape(1, _HID)
    ob1r = outb1.reshape(1, _HID)
    ob2r = outb2.reshape(1, _D)
    mapwt = mapW.T

    any_spec = pl.BlockSpec(memory_space=pl.ANY)
    vmem = pl.BlockSpec(memory_space=pltpu.VMEM)

    out = pl.pallas_call(
        _body,
        compiler_params=pltpu.CompilerParams(vmem_limit_bytes=63 << 20),
        in_specs=[any_spec, any_spec] + [vmem] * 15,
        out_specs=vmem,
        out_shape=jax.ShapeDtypeStruct((_B, _D), _F32),
        scratch_shapes=[
            pltpu.VMEM((_D, _D), _BF16),
            pltpu.VMEM((_BN, _D), _F32),
            pltpu.VMEM((_BN, _D), _F32),
            pltpu.VMEM((_D, _B), _F32),
            pltpu.VMEM((_D, _B), _BF16),
            pltpu.VMEM((_D, _B), _BF16),
            pltpu.VMEM((_D, _B), _F32),
            pltpu.VMEM((_D, _B), _F32),
            pltpu.SemaphoreType.DMA((_NSUB,)),
            pltpu.SemaphoreType.DMA((_NSUB,)),
        ],
    )(lap_down, lap_up, xt, t2, freqs, tW1, tb1r, tW2, tb2r, w0r, w1r,
      mapwt, mapbr, outW1, ob1r, outW2, ob2r)
    return out


# speedup vs baseline: 3.7353x; 1.2205x over previous
import math

import jax
import jax.numpy as jnp
from jax.experimental import pallas as pl
from jax.experimental.pallas import tpu as pltpu

_D = 4096
_B = 8
_HID = 64
_TDIM = 128
_BN = 256
_NB = _D // _BN
_NSUB = 4
_SUB = _BN // _NSUB
_DEPTH = 4
_F32 = jnp.float32
_BF16 = jnp.bfloat16


def _coef(w0, w1, i, j):
    return jnp.sum(w0[:, i:i + 1] * w1[:, j:j + 1], axis=0, keepdims=True)


def _body(ld_hbm, lu_hbm, xt_ref, t_ref, freqs_ref, tw1_ref, tb1_ref,
          tw2_ref, tb2_ref, w0_ref, w1_ref, mapwt_ref, mapb_ref,
          ow1_ref, ob1_ref, ow2_ref, ob2_ref,
          out_ref,
          stash_ref, bufa_ref, bufb_ref, bufc_ref, bufd_ref,
          u_ref, r_ref, p_ref, q_ref,
          xo_ref, sema, semb, semc, semd):
    bufs = [bufa_ref, bufb_ref, bufc_ref, bufd_ref]
    sems = [sema, semb, semc, semd]

    def start(src, chunk, buf, sem):
        for s in range(_NSUB):
            pltpu.make_async_copy(
                src.at[pl.ds(chunk * _BN + s * _SUB, _SUB), :],
                buf.at[pl.ds(s * _SUB, _SUB), :], sem.at[s]).start()

    def wait(src, chunk, buf, sem):
        for s in range(_NSUB):
            pltpu.make_async_copy(
                src.at[pl.ds(chunk * _BN + s * _SUB, _SUB), :],
                buf.at[pl.ds(s * _SUB, _SUB), :], sem.at[s]).wait()

    for c in range(_DEPTH):
        start(ld_hbm, c, bufs[c], sems[c])

    args = t_ref[...] * freqs_ref[...]
    cosr = jnp.cos(args)
    sinr = jnp.sin(args)
    h = jnp.dot(cosr, tw1_ref[0:_TDIM // 2, :], preferred_element_type=_F32)
    h = h + jnp.dot(sinr, tw1_ref[_TDIM // 2:_TDIM, :], preferred_element_type=_F32)
    h = h + tb1_ref[...]
    h = h * jax.lax.logistic(h)
    tout = jnp.dot(h, tw2_ref[...], preferred_element_type=_F32) + tb2_ref[...]
    w0 = w0_ref[...]
    w1 = w1_ref[...]
    c1 = _coef(w0, w1, 0, 0)
    c2 = _coef(w0, w1, 1, 0) + _coef(w0, w1, 0, 1)
    c3 = _coef(w0, w1, 2, 0) + _coef(w0, w1, 0, 2)
    c4 = _coef(w0, w1, 1, 1)
    c5 = _coef(w0, w1, 2, 1)
    c6 = _coef(w0, w1, 1, 2)
    c7 = _coef(w0, w1, 2, 2)

    xt = xt_ref[...]

    def consume0(i, buf):
        rows = pl.ds(pl.multiple_of(i * _BN, _BN), _BN)
        blk = buf[...]
        u_ref[rows, :] = jnp.dot(blk, xt,
                                 preferred_element_type=_F32).astype(_BF16)
        stash_ref[rows, :] = blk.astype(_BF16)

    def consume1(i, buf):
        rows = pl.ds(pl.multiple_of(i * _BN, _BN), _BN)
        v = jnp.dot(buf[...], xt, preferred_element_type=_F32)
        u = u_ref[rows, :].astype(_F32)
        r_ref[rows, :] = (c2 * u + c3 * v).astype(_BF16)
        p_ref[rows, :] = (c4 * u + c5 * v).astype(_BF16)
        q_ref[rows, :] = c6 * u + c7 * v

    def consume2(i, buf):
        rows = pl.ds(pl.multiple_of(i * _BN, _BN), _BN)
        sd = jnp.dot(stash_ref[rows, :], p_ref[...],
                     preferred_element_type=_F32)
        su = jnp.dot(buf[...], q_ref[...], preferred_element_type=_F32)
        xo_ref[rows, :] = (c1 * xt_ref[rows, :]
                           + r_ref[rows, :].astype(_F32) + sd + su)

    def phase(src, nxt_src, consume, last):
        def body(k, carry):
            for j in range(_DEPTH):
                i = pl.multiple_of(_DEPTH * k, _DEPTH) + j
                wait(src, i, bufs[j], sems[j])
                consume(i, bufs[j])
                nxt = i + _DEPTH

                @pl.when(nxt < _NB)
                def _pref():
                    start(src, nxt, bufs[j], sems[j])

                if not last:
                    @pl.when(nxt >= _NB)
                    def _pref2():
                        start(nxt_src, nxt - _NB, bufs[j], sems[j])
            return carry

        jax.lax.fori_loop(0, _NB // _DEPTH, body, 0, unroll=False)

    phase(ld_hbm, lu_hbm, consume0, last=False)
    phase(lu_hbm, lu_hbm, consume1, last=False)
    phase(lu_hbm, lu_hbm, consume2, last=True)

    x2t = jnp.dot(mapwt_ref[...], xo_ref[...],
                  preferred_element_type=_F32)
    h = jnp.transpose(x2t) + mapb_ref[...] + tout
    h = jnp.dot(h, ow1_ref[...], preferred_element_type=_F32) + ob1_ref[...]
    h = h * jax.lax.logistic(h)
    out_ref[...] = jnp.dot(h, ow2_ref[...], preferred_element_type=_F32) + ob2_ref[...]


def kernel(x, t, lap_down, lap_up, tW1, tb1, tW2, tb2, snn_w0, snn_w1,
           mapW, mapb, outW1, outb1, outW2, outb2):
    xt = x.T
    t2 = t.reshape(_B, 1)
    half = _TDIM // 2
    freqs = jnp.exp(
        -math.log(10000.0) * jnp.arange(0, half, dtype=_F32) / half
    ).reshape(1, half)
    w0r = snn_w0[0]
    w1r = snn_w1[:, 0, :]
    tb1r = tb1.reshape(1, _HID)
    tb2r = tb2.reshape(1, _HID)
    mapbr = mapb.reshape(1, _HID)
    ob1r = outb1.reshape(1, _HID)
    ob2r = outb2.reshape(1, _D)
    mapwt = mapW.T

    any_spec = pl.BlockSpec(memory_space=pl.ANY)
    vmem = pl.BlockSpec(memory_space=pltpu.VMEM)

    out = pl.pallas_call(
        _body,
        compiler_params=pltpu.CompilerParams(vmem_limit_bytes=63 << 20),
        in_specs=[any_spec, any_spec] + [vmem] * 15,
        out_specs=vmem,
        out_shape=jax.ShapeDtypeStruct((_B, _D), _F32),
        scratch_shapes=[
            pltpu.VMEM((_D, _D), _BF16),
            pltpu.VMEM((_BN, _D), _F32),
            pltpu.VMEM((_BN, _D), _F32),
            pltpu.VMEM((_BN, _D), _F32),
            pltpu.VMEM((_BN, _D), _F32),
            pltpu.VMEM((_D, _B), _BF16),
            pltpu.VMEM((_D, _B), _BF16),
            pltpu.VMEM((_D, _B), _BF16),
            pltpu.VMEM((_D, _B), _F32),
            pltpu.VMEM((_D, _B), _F32),
            pltpu.SemaphoreType.DMA((_NSUB,)),
            pltpu.SemaphoreType.DMA((_NSUB,)),
            pltpu.SemaphoreType.DMA((_NSUB,)),
            pltpu.SemaphoreType.DMA((_NSUB,)),
        ],
    )(lap_down, lap_up, xt, t2, freqs, tW1, tb1r, tW2, tb2r, w0r, w1r,
      mapwt, mapbr, outW1, ob1r, outW2, ob2r)
    return out
